# baseline (speedup 1.0000x reference)
import jax
import jax.numpy as jnp
from jax import lax
from jax.experimental import pallas as pl
from jax.experimental.pallas import tpu as pltpu
from jax.experimental.pallas import tpu_sc as plsc

NFEAT = 26
BATCH = 16384
DIM = 32
NC = 2
NS = 16
NW = NC * NS
BPW = BATCH // NW


def _body(*refs):
    idx_refs = refs[:NFEAT]
    tab_refs = refs[NFEAT:2 * NFEAT]
    out_hbm = refs[2 * NFEAT]
    idx_v, rows_v, sem = refs[2 * NFEAT + 1:]
    wid = lax.axis_index("s") * NC + lax.axis_index("c")
    base = wid * BPW
    for f in range(NFEAT):
        pltpu.sync_copy(idx_refs[f].at[pl.ds(base, BPW)], idx_v)
        pltpu.async_copy(tab_refs[f].at[idx_v], rows_v, sem).wait()
        pltpu.sync_copy(rows_v, out_hbm.at[pl.ds(base, BPW), pl.ds(f * DIM, DIM)])


def kernel(f00, f01, f02, f03, f04, f05, f06, f07, f08, f09, f10, f11, f12, f13, f14, f15, f16, f17, f18, f19, f20, f21, f22, f23, f24, f25, W_f00, W_f01, W_f02, W_f03, W_f04, W_f05, W_f06, W_f07, W_f08, W_f09, W_f10, W_f11, W_f12, W_f13, W_f14, W_f15, W_f16, W_f17, W_f18, W_f19, W_f20, W_f21, W_f22, W_f23, W_f24, W_f25):
    idxs = [jnp.asarray(x, jnp.int32) for x in (
        f00, f01, f02, f03, f04, f05, f06, f07, f08, f09, f10, f11, f12,
        f13, f14, f15, f16, f17, f18, f19, f20, f21, f22, f23, f24, f25)]
    tabs = [W_f00, W_f01, W_f02, W_f03, W_f04, W_f05, W_f06, W_f07, W_f08,
            W_f09, W_f10, W_f11, W_f12, W_f13, W_f14, W_f15, W_f16, W_f17,
            W_f18, W_f19, W_f20, W_f21, W_f22, W_f23, W_f24, W_f25]
    mesh = plsc.VectorSubcoreMesh(
        core_axis_name="c", subcore_axis_name="s", num_cores=NC, num_subcores=NS)
    run = pl.kernel(
        _body,
        out_type=jax.ShapeDtypeStruct((BATCH, NFEAT * DIM), jnp.float32),
        mesh=mesh,
        compiler_params=pltpu.CompilerParams(use_tc_tiling_on_sc=False),
        scratch_types=[
            pltpu.VMEM((BPW,), jnp.int32),
            pltpu.VMEM((BPW, DIM), jnp.float32),
            pltpu.SemaphoreType.DMA,
        ],
    )
    return run(*idxs, *tabs)


# speedup vs baseline: 1.0267x; 1.0267x over previous
import jax
import jax.numpy as jnp
from jax import lax
from jax.experimental import pallas as pl
from jax.experimental.pallas import tpu as pltpu
from jax.experimental.pallas import tpu_sc as plsc

NFEAT = 26
BATCH = 16384
DIM = 32
NC = 2
NS = 16
NW = NC * NS
BPW = BATCH // NW


NBUF = 4


def _body(*refs):
    idx_refs = refs[:NFEAT]
    tab_refs = refs[NFEAT:2 * NFEAT]
    out_hbm = refs[2 * NFEAT]
    rest = refs[2 * NFEAT + 1:]
    idx_all = rest[0]
    bufs = rest[1:1 + NBUF]
    gsems = rest[1 + NBUF:1 + 2 * NBUF]
    wsems = rest[1 + 2 * NBUF:1 + 3 * NBUF]
    isem = rest[1 + 3 * NBUF]
    wid = lax.axis_index("s") * NC + lax.axis_index("c")
    base = wid * BPW

    ih = [pltpu.async_copy(idx_refs[f].at[pl.ds(base, BPW)], idx_all.at[f], isem)
          for f in range(NFEAT)]
    for h in ih:
        h.wait()

    hg = [None] * NBUF
    hw = [None] * NBUF
    for f in range(NFEAT):
        s = f % NBUF
        if f >= NBUF:
            hw[s].wait()
        hg[s] = pltpu.async_copy(tab_refs[f].at[idx_all.at[f]], bufs[s], gsems[s])
        if f >= NBUF - 1:
            fp = f - (NBUF - 1)
            sp = fp % NBUF
            hg[sp].wait()
            hw[sp] = pltpu.async_copy(
                bufs[sp], out_hbm.at[pl.ds(base, BPW), pl.ds(fp * DIM, DIM)],
                wsems[sp])
    for fp in range(NFEAT - (NBUF - 1), NFEAT):
        sp = fp % NBUF
        hg[sp].wait()
        hw[sp] = pltpu.async_copy(
            bufs[sp], out_hbm.at[pl.ds(base, BPW), pl.ds(fp * DIM, DIM)],
            wsems[sp])
    for sp in set(fp % NBUF for fp in range(NFEAT - NBUF, NFEAT)):
        hw[sp].wait()


def kernel(f00, f01, f02, f03, f04, f05, f06, f07, f08, f09, f10, f11, f12, f13, f14, f15, f16, f17, f18, f19, f20, f21, f22, f23, f24, f25, W_f00, W_f01, W_f02, W_f03, W_f04, W_f05, W_f06, W_f07, W_f08, W_f09, W_f10, W_f11, W_f12, W_f13, W_f14, W_f15, W_f16, W_f17, W_f18, W_f19, W_f20, W_f21, W_f22, W_f23, W_f24, W_f25):
    idxs = [jnp.asarray(x, jnp.int32) for x in (
        f00, f01, f02, f03, f04, f05, f06, f07, f08, f09, f10, f11, f12,
        f13, f14, f15, f16, f17, f18, f19, f20, f21, f22, f23, f24, f25)]
    tabs = [W_f00, W_f01, W_f02, W_f03, W_f04, W_f05, W_f06, W_f07, W_f08,
            W_f09, W_f10, W_f11, W_f12, W_f13, W_f14, W_f15, W_f16, W_f17,
            W_f18, W_f19, W_f20, W_f21, W_f22, W_f23, W_f24, W_f25]
    mesh = plsc.VectorSubcoreMesh(
        core_axis_name="c", subcore_axis_name="s", num_cores=NC, num_subcores=NS)
    run = pl.kernel(
        _body,
        out_type=jax.ShapeDtypeStruct((BATCH, NFEAT * DIM), jnp.float32),
        mesh=mesh,
        compiler_params=pltpu.CompilerParams(use_tc_tiling_on_sc=False),
        scratch_types=(
            [pltpu.VMEM((NFEAT, BPW), jnp.int32)]
            + [pltpu.VMEM((BPW, DIM), jnp.float32) for _ in range(NBUF)]
            + [pltpu.SemaphoreType.DMA for _ in range(2 * NBUF + 1)]
        ),
    )
    return run(*idxs, *tabs)
